# baseline (speedup 1.0000x reference)
import functools

import jax
import jax.numpy as jnp
from jax import lax
from jax.experimental import pallas as pl
from jax.experimental.pallas import tpu as pltpu
from jax.experimental.pallas import tpu_sc as plsc

_VOCAB = 100000
_D = 768
_B = 4
_S = 8192
_BS = _B * _S

_NW = 32
_SPW = _S // _NW
_NG = 64
_C = _SPW // _NG
_NCH = _NG * _B
_LG = _D // 16


@functools.partial(
    pl.kernel,
    mesh=plsc.VectorSubcoreMesh(core_axis_name="c", subcore_axis_name="s"),
    out_type=jax.ShapeDtypeStruct((_BS, _D), jnp.float32),
    scratch_types=[
        pltpu.VMEM((_B, _NG, _C), jnp.int32),
        pltpu.VMEM((2, _B, _C, _D), jnp.float32),
        pltpu.VMEM((2, _C, _D), jnp.float32),
        pltpu.SemaphoreType.DMA,
        pltpu.SemaphoreType.DMA,
        pltpu.SemaphoreType.DMA,
    ],
)
def _emb_lookup(x_hbm, tok_hbm, pos_hbm, out_hbm, idx_v, tokb, posb,
                psem, gsem, osem):
    cid = lax.axis_index("c")
    sid = lax.axis_index("s")
    wid = sid * 2 + cid
    s_base = wid * _SPW

    for b in range(_B):
        pltpu.sync_copy(x_hbm.at[b, wid], idx_v.at[b])

    def issue_group(g, par):
        pltpu.async_copy(pos_hbm.at[pl.ds(s_base + g * _C, _C)],
                         posb.at[par], psem)
        for b in range(_B):
            pltpu.async_copy(tok_hbm.at[idx_v.at[b, g]],
                             tokb.at[par, b], gsem)

    def drain(sem, shaped):
        pltpu.make_async_copy(pos_hbm.at[pl.ds(0, _C)], shaped, sem).wait()

    issue_group(0, 0)

    def consume(g, par):
        @pl.when(g + 1 < _NG)
        def _():
            @pl.when(g >= 1)
            def _():
                for b in range(_B):
                    drain(osem, tokb.at[1 - par, b])
            issue_group(g + 1, 1 - par)

        drain(psem, posb.at[par])
        for b in range(_B):
            drain(gsem, tokb.at[par, b])

        def row_body(r, c2):
            for k in range(_LG):
                sl = pl.ds(k * 16, 16)
                pv = posb[par, r, sl]
                for b in range(_B):
                    plsc.addupdate(tokb.at[par, b, r, sl], pv)
            return c2

        lax.fori_loop(0, _C, row_body, 0, unroll=False)
        for b in range(_B):
            row = b * _S + s_base + g * _C
            pltpu.async_copy(tokb.at[par, b], out_hbm.at[pl.ds(row, _C)],
                             osem)

    def pair_body(gg, carry):
        consume(2 * gg, 0)
        consume(2 * gg + 1, 1)
        return carry

    lax.fori_loop(0, _NG // 2, pair_body, 0, unroll=False)
    for par in range(2):
        for b in range(_B):
            drain(osem, tokb.at[par, b])


def kernel(x, token_table, pos_table):
    xr = x.astype(jnp.int32).reshape(_B, _NW, _NG, _C)
    out = _emb_lookup(xr, token_table, pos_table)
    return out.reshape(_B, _S, _D)


# speedup vs baseline: 1.1482x; 1.1482x over previous
import functools

import jax
import jax.numpy as jnp
from jax import lax
from jax.experimental import pallas as pl
from jax.experimental.pallas import tpu as pltpu
from jax.experimental.pallas import tpu_sc as plsc

_VOCAB = 100000
_D = 768
_B = 4
_S = 8192
_BS = _B * _S

_NW = 32
_SPW = _S // _NW
_NG = 32
_C = _SPW // _NG
_NCH = _NG * _B
_LG = _D // 16


@functools.partial(
    pl.kernel,
    mesh=plsc.VectorSubcoreMesh(core_axis_name="c", subcore_axis_name="s"),
    out_type=jax.ShapeDtypeStruct((_BS, _D), jnp.float32),
    scratch_types=[
        pltpu.VMEM((_B, _NG, _C), jnp.int32),
        pltpu.VMEM((2, _B, _C, _D), jnp.float32),
        pltpu.VMEM((2, _C, _D), jnp.float32),
        pltpu.SemaphoreType.DMA,
        pltpu.SemaphoreType.DMA,
        pltpu.SemaphoreType.DMA,
    ],
)
def _emb_lookup(x_hbm, tok_hbm, pos_hbm, out_hbm, idx_v, tokb, posb,
                psem, gsem, osem):
    cid = lax.axis_index("c")
    sid = lax.axis_index("s")
    wid = sid * 2 + cid
    s_base = wid * _SPW

    def issue_group(g, par):
        pltpu.async_copy(pos_hbm.at[pl.ds(s_base + g * _C, _C)],
                         posb.at[par], psem)
        for b in range(_B):
            pltpu.async_copy(tok_hbm.at[idx_v.at[b, g]],
                             tokb.at[par, b], gsem)

    def drain(sem, shaped):
        pltpu.make_async_copy(pos_hbm.at[pl.ds(0, _C)], shaped, sem).wait()

    pltpu.async_copy(pos_hbm.at[pl.ds(s_base, _C)], posb.at[0], psem)
    for b in range(_B):
        pltpu.async_copy(x_hbm.at[b, wid], idx_v.at[b], osem)
    drain(osem, idx_v)

    for b in range(_B):
        pltpu.async_copy(tok_hbm.at[idx_v.at[b, 0]], tokb.at[0, b], gsem)

    def consume(g, par):
        @pl.when(g + 1 < _NG)
        def _():
            @pl.when(g >= 1)
            def _():
                for b in range(_B):
                    drain(osem, tokb.at[1 - par, b])
            issue_group(g + 1, 1 - par)

        drain(psem, posb.at[par])
        for b in range(_B):
            drain(gsem, tokb.at[par, b])

        def row_body(r, c2):
            for k in range(_LG):
                sl = pl.ds(k * 16, 16)
                pv = posb[par, r, sl]
                for b in range(_B):
                    plsc.addupdate(tokb.at[par, b, r, sl], pv)
            return c2

        lax.fori_loop(0, _C, row_body, 0, unroll=False)
        for b in range(_B):
            row = b * _S + s_base + g * _C
            pltpu.async_copy(tokb.at[par, b], out_hbm.at[pl.ds(row, _C)],
                             osem)

    def pair_body(gg, carry):
        consume(2 * gg, 0)
        consume(2 * gg + 1, 1)
        return carry

    lax.fori_loop(0, _NG // 2, pair_body, 0, unroll=False)
    for par in range(2):
        for b in range(_B):
            drain(osem, tokb.at[par, b])


def kernel(x, token_table, pos_table):
    xr = x.astype(jnp.int32).reshape(_B, _NW, _NG, _C)
    out = _emb_lookup(xr, token_table, pos_table)
    return out.reshape(_B, _S, _D)
